# baseline (speedup 1.0000x reference)
import jax
import jax.numpy as jnp
from jax.experimental import pallas as pl
from jax.experimental.pallas import tpu as pltpu

_HEAD = 8


def _body(mt_hbm, vpad_hbm, out_hbm, head_v, val_v, sem_big, sem_h, sem_v, sem_o):
    n = mt_hbm.shape[0]
    big = pltpu.make_async_copy(
        mt_hbm.at[pl.ds(_HEAD, n - _HEAD)],
        out_hbm.at[pl.ds(_HEAD, n - _HEAD)],
        sem_big,
    )
    big.start()
    h = pltpu.make_async_copy(mt_hbm.at[pl.ds(0, _HEAD)], head_v, sem_h)
    h.start()
    v = pltpu.make_async_copy(vpad_hbm, val_v, sem_v)
    v.start()
    h.wait()
    v.wait()
    head_v[...] = head_v[...] + val_v[...]
    o = pltpu.make_async_copy(head_v, out_hbm.at[pl.ds(0, _HEAD)], sem_o)
    o.start()
    o.wait()
    big.wait()


def kernel(main_tensor, value):
    n, d = main_tensor.shape
    vpad = jnp.zeros((_HEAD, d), dtype=value.dtype).at[: value.shape[0]].set(value)
    return pl.pallas_call(
        _body,
        out_shape=jax.ShapeDtypeStruct((n, d), main_tensor.dtype),
        in_specs=[
            pl.BlockSpec(memory_space=pltpu.HBM),
            pl.BlockSpec(memory_space=pltpu.HBM),
        ],
        out_specs=pl.BlockSpec(memory_space=pltpu.HBM),
        scratch_shapes=[
            pltpu.VMEM((_HEAD, d), main_tensor.dtype),
            pltpu.VMEM((_HEAD, d), value.dtype),
            pltpu.SemaphoreType.DMA,
            pltpu.SemaphoreType.DMA,
            pltpu.SemaphoreType.DMA,
            pltpu.SemaphoreType.DMA,
        ],
    )(main_tensor, vpad)


# speedup vs baseline: 11.6924x; 11.6924x over previous
import jax
import jax.numpy as jnp
from jax.experimental import pallas as pl
from jax.experimental.pallas import tpu as pltpu

_BLK = 4000


def _body(x_ref, v_ref, o_ref):
    o_ref[...] = x_ref[...]

    @pl.when(pl.program_id(0) == 0)
    def _():
        o_ref[0:1, :] += v_ref[...]


def kernel(main_tensor, value):
    n, d = main_tensor.shape
    x = main_tensor.reshape(n // 2, 2 * d)
    v = value.reshape(1, 2 * d)
    n2 = n // 2
    out = pl.pallas_call(
        _body,
        grid=(n2 // _BLK,),
        out_shape=jax.ShapeDtypeStruct((n2, 2 * d), main_tensor.dtype),
        in_specs=[
            pl.BlockSpec((_BLK, 2 * d), lambda i: (i, 0)),
            pl.BlockSpec((1, 2 * d), lambda i: (0, 0)),
        ],
        out_specs=pl.BlockSpec((_BLK, 2 * d), lambda i: (i, 0)),
    )(x, v)
    return out.reshape(n, d)


# speedup vs baseline: 16.1152x; 1.3783x over previous
import jax
import jax.numpy as jnp
from jax.experimental import pallas as pl
from jax.experimental.pallas import tpu as pltpu

_BLK = 8000


def _body(x_ref, v_ref, o_ref):
    o_ref[...] = x_ref[...]

    @pl.when(pl.program_id(0) == 0)
    def _():
        o_ref[0:8, :] += v_ref[...]


def kernel(main_tensor, value):
    n, d = main_tensor.shape
    vpad = jnp.zeros((8, d), dtype=value.dtype).at[: value.shape[0]].set(value)
    return pl.pallas_call(
        _body,
        grid=(n // _BLK,),
        out_shape=jax.ShapeDtypeStruct((n, d), main_tensor.dtype),
        in_specs=[
            pl.BlockSpec((_BLK, d), lambda i: (i, 0)),
            pl.BlockSpec((8, d), lambda i: (0, 0)),
        ],
        out_specs=pl.BlockSpec((_BLK, d), lambda i: (i, 0)),
    )(main_tensor, vpad)
